# Initial kernel scaffold; baseline (speedup 1.0000x reference)
import jax
import jax.numpy as jnp
from jax.experimental import pallas as pl


def kernel(features, rois, img_size):
    raise NotImplementedError("write your pallas kernel here")


# speedup vs baseline: 10.9445x; 10.9445x over previous
import functools

import jax
import jax.numpy as jnp
import numpy as np
from jax import lax
from jax.experimental import pallas as pl
from jax.experimental.pallas import tpu as pltpu
from jax.experimental.pallas import tpu_sc as plsc

POOL = 7
SUB = 2
HS = POOL * SUB
WS = POOL * SUB
L = 16
NPTS = HS * WS
NVEC = (NPTS + L - 1) // L
PPAD = NVEC * L
NOUT = POOL * POOL
KOUT = 4

N_CG = 8
N_RG = 4
CPG = 16
RPG = 256
CHUNK = 16
NCHUNK = RPG // CHUNK


def _point_tables():
    p = np.arange(PPAD)
    i = np.where(p < NPTS, p // WS, 0).astype(np.float32)
    j = np.where(p < NPTS, p % WS, 0).astype(np.float32)
    o = np.arange(KOUT * L)
    oi = np.where(o < NOUT, o // POOL, 0)
    oj = np.where(o < NOUT, o % POOL, 0)
    tbl = []
    for di in range(SUB):
        for dj in range(SUB):
            tbl.append((SUB * oi + di) * WS + (SUB * oj + dj))
    pool = np.stack(tbl).astype(np.int32)
    return i, j, pool


_I_OF_P, _J_OF_P, _POOL_TBL = _point_tables()


def _sc_body(H, W, feat_hbm, rois_hbm, sv_hbm, iv_hbm, jv_hbm, pool_hbm,
             out_hbm, feat_v, rois_v, sv_v, iv_v, jv_v, pool_v, samp_v,
             out_v):
    wid = lax.axis_index("s") * 2 + lax.axis_index("c")
    cg = wid % N_CG
    rg = wid // N_CG

    pltpu.sync_copy(feat_hbm.at[cg], feat_v)
    pltpu.sync_copy(rois_hbm.at[rg], rois_v)
    pltpu.sync_copy(sv_hbm, sv_v)
    pltpu.sync_copy(iv_hbm, iv_v)
    pltpu.sync_copy(jv_hbm, jv_v)
    pltpu.sync_copy(pool_hbm, pool_v)

    sy = sv_v[pl.ds(0, L)]
    sx = sv_v[pl.ds(L, L)]
    pool_idx = [[pool_v[pl.ds((d * KOUT + k) * L, L)] for k in range(KOUT)]
                for d in range(SUB * SUB)]
    inv = jnp.float32(1.0 / HS)
    rflat = rois_v
    fflat = feat_v
    sflat = samp_v
    row_off = [jnp.full((L,), d * RPG, jnp.int32) for d in range(4)]
    c_feat = [jnp.full((L,), c * H * W, jnp.int32) for c in range(CPG)]
    c_samp = [jnp.full((L,), c * PPAD, jnp.int32) for c in range(CPG)]

    def roi_body(r, _):
        rvec = jnp.full((L,), r, jnp.int32)
        rif0 = plsc.load_gather(rflat, [row_off[0] + rvec]) * sy
        rif1 = plsc.load_gather(rflat, [row_off[1] + rvec]) * sx
        rif2 = plsc.load_gather(rflat, [row_off[2] + rvec]) * sy
        rif3 = plsc.load_gather(rflat, [row_off[3] + rvec]) * sx
        h_step = (rif2 - rif0) * inv
        w_step = (rif3 - rif1) * inv

        def vec_body(v, _):
            iv = iv_v[pl.ds(v * L, L)]
            jv = jv_v[pl.ds(v * L, L)]
            yc = (iv + 0.5) * h_step + rif0
            xc = (jv + 0.5) * w_step + rif1
            iy0 = yc.astype(jnp.int32)
            ix0 = xc.astype(jnp.int32)
            fy = yc - iy0.astype(jnp.float32)
            fx = xc - ix0.astype(jnp.float32)
            iy1 = iy0 + (fy > 0.0).astype(jnp.int32)
            ix1 = ix0 + (fx > 0.0).astype(jnp.int32)
            b00 = iy0 * W + ix0
            b01 = iy0 * W + ix1
            b10 = iy1 * W + ix0
            b11 = iy1 * W + ix1
            wy1 = fy
            wy0 = 1.0 - fy
            wx1 = fx
            wx0 = 1.0 - fx
            w00 = wy0 * wx0
            w01 = wy0 * wx1
            w10 = wy1 * wx0
            w11 = wy1 * wx1
            for c in range(CPG):
                co = c_feat[c]
                val = (plsc.load_gather(fflat, [co + b00]) * w00
                       + plsc.load_gather(fflat, [co + b01]) * w01
                       + plsc.load_gather(fflat, [co + b10]) * w10
                       + plsc.load_gather(fflat, [co + b11]) * w11)
                sflat[pl.ds(c * PPAD + v * L, L)] = val
            return 0

        lax.fori_loop(0, NVEC, vec_body, 0)

        for c in range(CPG):
            co = c_samp[c]
            for k in range(KOUT):
                g0 = plsc.load_gather(sflat, [co + pool_idx[0][k]])
                g1 = plsc.load_gather(sflat, [co + pool_idx[1][k]])
                g2 = plsc.load_gather(sflat, [co + pool_idx[2][k]])
                g3 = plsc.load_gather(sflat, [co + pool_idx[3][k]])
                m = jnp.maximum(jnp.maximum(g0, g1), jnp.maximum(g2, g3))
                out_v[pl.ds(c * (CHUNK * KOUT * L) + (r % CHUNK) * (KOUT * L)
                            + k * L, L)] = m
        return 0

    def chunk_body(ch, _):
        lax.fori_loop(ch * CHUNK, (ch + 1) * CHUNK, roi_body, 0)
        pltpu.sync_copy(
            out_v,
            out_hbm.at[wid, pl.ds(ch * (CPG * CHUNK * KOUT * L),
                                  CPG * CHUNK * KOUT * L)])
        return 0

    lax.fori_loop(0, NCHUNK, chunk_body, 0)


def kernel(features, rois, img_size):
    fs = features.shape
    C = fs[0] * fs[1]
    H, W = fs[2], fs[3]
    n = rois.shape[0]

    feat = features.reshape(N_CG, CPG * H * W)
    npad = N_RG * RPG
    rois_p = jnp.pad(rois, ((0, npad - n), (0, 0))).T
    rois_p = rois_p.reshape(4, N_RG, RPG).transpose(1, 0, 2).reshape(
        N_RG, 4 * RPG)
    sy = (H - 1.0) / (img_size[0] - 1.0)
    sx = (W - 1.0) / (img_size[1] - 1.0)
    sv = jnp.concatenate([jnp.broadcast_to(sy, (L,)),
                          jnp.broadcast_to(sx, (L,))]).astype(jnp.float32)
    iv = jnp.asarray(_I_OF_P)
    jv = jnp.asarray(_J_OF_P)
    pool = jnp.asarray(_POOL_TBL).reshape(-1)

    words_per_tile = NCHUNK * CPG * CHUNK * KOUT * L
    mesh = plsc.VectorSubcoreMesh(core_axis_name="c", subcore_axis_name="s",
                                  num_cores=2, num_subcores=16)
    run = pl.kernel(
        functools.partial(_sc_body, H, W),
        out_type=jax.ShapeDtypeStruct((N_CG * N_RG, words_per_tile),
                                      jnp.float32),
        mesh=mesh,
        scratch_types=[
            pltpu.VMEM((CPG * H * W,), jnp.float32),
            pltpu.VMEM((4 * RPG,), jnp.float32),
            pltpu.VMEM((2 * L,), jnp.float32),
            pltpu.VMEM((PPAD,), jnp.float32),
            pltpu.VMEM((PPAD,), jnp.float32),
            pltpu.VMEM((SUB * SUB * KOUT * L,), jnp.int32),
            pltpu.VMEM((CPG * PPAD,), jnp.float32),
            pltpu.VMEM((CPG * CHUNK * KOUT * L,), jnp.float32),
        ],
        compiler_params=pltpu.CompilerParams(needs_layout_passes=False),
    )
    out = run(feat, rois_p, sv, iv, jv, pool)

    arr = out.reshape(N_RG, N_CG, NCHUNK, CPG, CHUNK, KOUT * L)
    arr = arr.transpose(1, 3, 0, 2, 4, 5).reshape(C, npad, KOUT * L)
    arr = arr[:, :n, :NOUT].reshape(C, n, POOL, POOL)
    return arr.transpose(1, 0, 2, 3)
